# Initial kernel scaffold; baseline (speedup 1.0000x reference)
import jax
import jax.numpy as jnp
from jax.experimental import pallas as pl


def kernel(tensor, ROI):
    raise NotImplementedError("write your pallas kernel here")


# speedup vs baseline: 1.3012x; 1.3012x over previous
import jax
import jax.numpy as jnp
from jax import lax
from jax.experimental import pallas as pl
from jax.experimental.pallas import tpu as pltpu

FEA = 14
OUT = 7
SCALE = 1.0 / 16.0


def _roi_kernel(coords_ref, t_ref, o_ref):
    n = pl.program_id(0)
    x1 = coords_ref[4 * n + 0]
    y1 = coords_ref[4 * n + 1]
    x2 = coords_ref[4 * n + 2]
    y2 = coords_ref[4 * n + 3]

    ij = lax.broadcasted_iota(jnp.int32, (OUT * OUT, FEA * FEA), 0)
    yx = lax.broadcasted_iota(jnp.int32, (OUT * OUT, FEA * FEA), 1)
    i = ij // OUT
    j = ij - OUT * i
    y = yx // FEA
    x = yx - FEA * y

    def axis_w(pos, q, a, b):
        L = b - a
        s = a + (q * L) // OUT
        e = a + ((q + 1) * L + (OUT - 1)) // OUT
        m = (pos >= s) & (pos < e)
        d = jnp.maximum(e - s, 1).astype(jnp.float32)
        return m.astype(jnp.float32) / d

    w = axis_w(y, i, y1, y2) * axis_w(x, j, x1, x2)
    o_ref[0] = jnp.dot(w, t_ref[...], preferred_element_type=jnp.float32)


def kernel(tensor, ROI):
    B, C, H, W = tensor.shape
    N = ROI.shape[0]
    t = tensor.reshape(B * C, H * W).T

    c = ROI[:, 1:] * SCALE
    x1 = jnp.clip(jnp.floor(c[:, 0]), 0, FEA)
    y1 = jnp.clip(jnp.floor(c[:, 1]), 0, FEA)
    x2 = jnp.clip(jnp.ceil(c[:, 2]), 0, FEA)
    y2 = jnp.clip(jnp.ceil(c[:, 3]), 0, FEA)
    coords = jnp.stack([x1, y1, x2, y2], axis=1).astype(jnp.int32).reshape(-1)

    out = pl.pallas_call(
        _roi_kernel,
        out_shape=jax.ShapeDtypeStruct((N, OUT * OUT, B * C), jnp.float32),
        grid_spec=pltpu.PrefetchScalarGridSpec(
            num_scalar_prefetch=1,
            grid=(N,),
            in_specs=[
                pl.BlockSpec((H * W, B * C), lambda n, s: (0, 0)),
            ],
            out_specs=pl.BlockSpec((1, OUT * OUT, B * C), lambda n, s: (n, 0, 0)),
        ),
        compiler_params=pltpu.CompilerParams(
            dimension_semantics=("parallel",),
        ),
        name="roi_adaptive_pool",
    )(coords, t)

    return out.transpose(0, 2, 1).reshape(N * B, C, OUT, OUT)
